# Initial kernel scaffold; baseline (speedup 1.0000x reference)
import jax
import jax.numpy as jnp
from jax.experimental import pallas as pl


def kernel(x, baseline, shared_W1, shared_b1, shared_W2, shared_b2, Wr_sp, br_sp, Wr_sc, br_sc, sp_W1, sp_b1, sp_W2, sp_b2, sc_W1, sc_b1, sc_W2, sc_b2):
    raise NotImplementedError("write your pallas kernel here")


# speedup vs baseline: 1.1874x; 1.1874x over previous
import functools
import math

import jax
import jax.numpy as jnp
from jax import lax
from jax.experimental import pallas as pl
from jax.experimental.pallas import tpu as pltpu

B, C, T, D = 64, 8, 128, 256
FF = 1024
E = 8
CAPF = 1.25
_CAP = float(math.ceil(CAPF * B / E))
CT = C * T


def _means_logits_body(x_ref, bl_ref, wr_ref, br_ref, out_ref):
    inv = 1.0 / CT
    m_x = jnp.sum(x_ref[0], axis=0, keepdims=True) * inv
    m_b = jnp.sum(bl_ref[0], axis=0, keepdims=True) * inv
    m_d = m_x - m_b
    w_b = wr_ref[:, 0:D]
    w_x = wr_ref[:, D:2 * D]
    w_d = wr_ref[:, 2 * D:3 * D]
    dn = (((1,), (1,)), ((), ()))
    logits = (
        lax.dot_general(m_b, w_b, dn, preferred_element_type=jnp.float32)
        + lax.dot_general(m_x, w_x, dn, preferred_element_type=jnp.float32)
        + lax.dot_general(m_d, w_d, dn, preferred_element_type=jnp.float32)
        + br_ref[...]
    )
    out_ref[0] = logits


def _routing_body(lg_ref, eidx_sp_ref, eidx_sc_ref, scale_sp_ref, scale_sc_ref):
    lg = lg_ref[:, 0, :]
    col = lax.broadcasted_iota(jnp.int32, (B, 2 * E), 1)
    is_sp = col < E
    neg = jnp.float32(-1e30)
    m_sp = jnp.max(jnp.where(is_sp, lg, neg), axis=1, keepdims=True)
    m_sc = jnp.max(jnp.where(is_sp, neg, lg), axis=1, keepdims=True)
    mm = jnp.where(is_sp, m_sp, m_sc)
    ex = jnp.exp(lg - mm)
    s_sp = jnp.sum(jnp.where(is_sp, ex, 0.0), axis=1, keepdims=True)
    s_sc = jnp.sum(jnp.where(is_sp, 0.0, ex), axis=1, keepdims=True)
    gate_sp = 1.0 / s_sp
    gate_sc = 1.0 / s_sc
    eq = lg == mm
    cand = jnp.where(eq, col, 2 * E)
    e_sp = jnp.min(jnp.where(is_sp, cand, 2 * E), axis=1, keepdims=True)
    e_sc = jnp.min(jnp.where(is_sp, 2 * E, cand), axis=1, keepdims=True)
    sel = jnp.where(is_sp, e_sp, e_sc)
    oh = (col == sel).astype(jnp.float32)
    tri = (lax.broadcasted_iota(jnp.int32, (B, B), 0)
           >= lax.broadcasted_iota(jnp.int32, (B, B), 1)).astype(jnp.float32)
    pos = lax.dot_general(tri, oh, (((1,), (0,)), ((), ())),
                          preferred_element_type=jnp.float32)
    pos_sp = jnp.sum(jnp.where(is_sp, pos * oh, 0.0), axis=1, keepdims=True)
    pos_sc = jnp.sum(jnp.where(is_sp, 0.0, pos * oh), axis=1, keepdims=True)
    keep_sp = (pos_sp <= _CAP).astype(jnp.float32)
    keep_sc = (pos_sc <= _CAP).astype(jnp.float32)
    eidx_sp_ref[...] = e_sp
    eidx_sc_ref[...] = e_sc - E
    scale_sp_ref[...] = gate_sp * keep_sp
    scale_sc_ref[...] = gate_sc * keep_sc


def _ffn_body(idx_sp_ref, idx_sc_ref, ssp_ref, ssc_ref,
              x_ref, sW1_ref, sb1_ref, sW2_ref, sb2_ref,
              pW1_ref, pb1_ref, pW2_ref, pb2_ref,
              cW1_ref, cb1_ref, cW2_ref, cb2_ref, out_ref):
    b = pl.program_id(0)
    xb = x_ref[0]
    dn = (((1,), (1,)), ((), ()))

    def ffn(w1, b1, w2, b2):
        h = lax.dot_general(xb, w1, dn, preferred_element_type=jnp.float32)
        h = jax.nn.gelu(h + b1)
        o = lax.dot_general(h, w2, dn, preferred_element_type=jnp.float32)
        return o + b2

    acc = ffn(sW1_ref[...], sb1_ref[...], sW2_ref[...], sb2_ref[...])
    e1 = idx_sp_ref[b]
    acc = acc + ssp_ref[b] * ffn(pW1_ref[e1], pb1_ref[e1], pW2_ref[e1], pb2_ref[e1])
    e2 = idx_sc_ref[b]
    acc = acc + ssc_ref[b] * ffn(cW1_ref[e2], cb1_ref[e2], cW2_ref[e2], cb2_ref[e2])
    out_ref[0] = acc


def kernel(x, baseline, shared_W1, shared_b1, shared_W2, shared_b2,
           Wr_sp, br_sp, Wr_sc, br_sc,
           sp_W1, sp_b1, sp_W2, sp_b2, sc_W1, sc_b1, sc_W2, sc_b2):
    x2 = x.reshape(B, CT, D)
    bl2 = baseline.reshape(B, CT, D)
    wr_cat = jnp.concatenate([Wr_sp, Wr_sc], axis=0)
    br_cat = jnp.concatenate([br_sp, br_sc], axis=0)[None, :]

    logits = pl.pallas_call(
        _means_logits_body,
        grid=(B,),
        in_specs=[
            pl.BlockSpec((1, CT, D), lambda b: (b, 0, 0)),
            pl.BlockSpec((1, CT, D), lambda b: (b, 0, 0)),
            pl.BlockSpec((2 * E, 3 * D), lambda b: (0, 0)),
            pl.BlockSpec((1, 2 * E), lambda b: (0, 0)),
        ],
        out_specs=pl.BlockSpec((1, 1, 2 * E), lambda b: (b, 0, 0)),
        out_shape=jax.ShapeDtypeStruct((B, 1, 2 * E), jnp.float32),
    )(x2, bl2, wr_cat, br_cat)

    eidx_sp, eidx_sc, scale_sp, scale_sc = pl.pallas_call(
        _routing_body,
        out_shape=(
            jax.ShapeDtypeStruct((B, 1), jnp.int32),
            jax.ShapeDtypeStruct((B, 1), jnp.int32),
            jax.ShapeDtypeStruct((B, 1), jnp.float32),
            jax.ShapeDtypeStruct((B, 1), jnp.float32),
        ),
    )(logits)

    full = lambda s: pl.BlockSpec(s, lambda b, *_: tuple(0 for _ in s))
    y2 = pl.pallas_call(
        _ffn_body,
        grid_spec=pltpu.PrefetchScalarGridSpec(
            num_scalar_prefetch=4,
            grid=(B,),
            in_specs=[
                pl.BlockSpec((1, CT, D), lambda b, *_: (b, 0, 0)),
                full((FF, D)), full((1, FF)), full((D, FF)), full((1, D)),
                full((E, FF, D)), full((E, FF)), full((E, D, FF)), full((E, D)),
                full((E, FF, D)), full((E, FF)), full((E, D, FF)), full((E, D)),
            ],
            out_specs=pl.BlockSpec((1, CT, D), lambda b, *_: (b, 0, 0)),
        ),
        out_shape=jax.ShapeDtypeStruct((B, CT, D), jnp.float32),
    )(eidx_sp.reshape(B), eidx_sc.reshape(B),
      scale_sp.reshape(B), scale_sc.reshape(B),
      x2, shared_W1, shared_b1[None, :], shared_W2, shared_b2[None, :],
      sp_W1, sp_b1, sp_W2, sp_b2, sc_W1, sc_b1, sc_W2, sc_b2)

    return y2.reshape(B, C, T, D)
